# baseline (speedup 1.0000x reference)
import functools

import jax
import jax.numpy as jnp
from jax import lax
from jax.experimental import pallas as pl
from jax.experimental.pallas import tpu as pltpu
from jax.experimental.pallas import tpu_sc as plsc

VOCAB = 1_000_000
DIM = 64
BATCH = 16384
NWORDS = 20

NC = 2
NS = 16
NWK = NC * NS
BPW = BATCH // NWK
CB = 32
NCHUNK = BPW // CB
NPAIR = CB * NWORDS
IDX_SPLIT = 128
NSPLIT = NPAIR // IDX_SPLIT
NBLK = 5
L = 16


def _w2v_body(cen_words, ctx_words, cen_tab, ctx_tab, out,
              cidx_v, xidx_v, crow_v, cpar_v, xrow_v, xpar_v,
              cen_v, ctx_v, cen_t, sco_v, sem):
    wid = lax.axis_index("s") * NC + lax.axis_index("c")
    base = wid * BPW
    iot = lax.iota(jnp.int32, L)

    def chunk(k, carry):
        cb = base + k * CB
        pltpu.sync_copy(cen_words.at[pl.ds(cb, CB)], cidx_v)
        pltpu.sync_copy(ctx_words.at[pl.ds(cb * NWORDS, NPAIR)], xidx_v)
        for t in range(CB // L):
            w = cidx_v[pl.ds(t * L, L)]
            crow_v[pl.ds(t * L, L)] = w >> 1
            cpar_v[pl.ds(t * L, L)] = w & 1
        for t in range(NPAIR // L):
            w = xidx_v[pl.ds(t * L, L)]
            xrow_v[pl.ds(t * L, L)] = w >> 1
            xpar_v[pl.ds(t * L, L)] = w & 1
        copies = [pltpu.async_copy(cen_tab.at[crow_v], cen_v, sem)]
        for j in range(NSPLIT):
            copies.append(pltpu.async_copy(
                ctx_tab.at[xrow_v.at[pl.ds(j * IDX_SPLIT, IDX_SPLIT)]],
                ctx_v.at[pl.ds(j * IDX_SPLIT, IDX_SPLIT)], sem))
        for c in copies:
            c.wait()
        for g in range(CB // L):
            bvec = iot + g * L
            colbase = cpar_v[pl.ds(g * L, L)] * DIM
            for d in range(DIM):
                v = plsc.load_gather(cen_v, [bvec, colbase + d])
                cen_t[d, pl.ds(g * L, L)] = v
        for g in range(CB // L):
            bvec = iot + g * L
            nbase = bvec * NWORDS
            for nb in range(NWORDS // NBLK):
                rows = [nbase + (nb * NBLK + q) for q in range(NBLK)]
                cols = [plsc.load_gather(xpar_v, [rows[q]]) * DIM
                        for q in range(NBLK)]
                accs0 = (jnp.zeros((L,), jnp.float32),) * NBLK

                def dstep(d, accs, rows=rows, cols=cols, g=g):
                    cenvd = cen_t[d, pl.ds(g * L, L)]
                    return tuple(
                        accs[q]
                        + plsc.load_gather(ctx_v, [rows[q], cols[q] + d])
                        * cenvd
                        for q in range(NBLK))

                accs = lax.fori_loop(0, DIM, dstep, accs0)
                for q in range(NBLK):
                    plsc.store_scatter(sco_v, [rows[q]], accs[q])
        pltpu.sync_copy(sco_v, out.at[pl.ds(cb * NWORDS, NPAIR)])
        return carry

    lax.fori_loop(0, NCHUNK, chunk, 0)


_w2v = functools.partial(
    pl.kernel,
    mesh=plsc.VectorSubcoreMesh(core_axis_name="c", subcore_axis_name="s"),
    compiler_params=pltpu.CompilerParams(needs_layout_passes=False),
    out_type=jax.ShapeDtypeStruct((BATCH * NWORDS,), jnp.float32),
    scratch_types=[
        pltpu.VMEM((CB,), jnp.int32),
        pltpu.VMEM((NPAIR,), jnp.int32),
        pltpu.VMEM((CB,), jnp.int32),
        pltpu.VMEM((CB,), jnp.int32),
        pltpu.VMEM((NPAIR,), jnp.int32),
        pltpu.VMEM((NPAIR,), jnp.int32),
        pltpu.VMEM((CB, 2 * DIM), jnp.float32),
        pltpu.VMEM((NPAIR, 2 * DIM), jnp.float32),
        pltpu.VMEM((DIM, CB), jnp.float32),
        pltpu.VMEM((NPAIR,), jnp.float32),
        pltpu.SemaphoreType.DMA,
    ],
)(_w2v_body)


@jax.jit
def kernel(center_words, context_words, center_table, context_table):
    ctx_flat = context_words.astype(jnp.int32).reshape(BATCH * NWORDS)
    cen128 = center_table.reshape(VOCAB // 2, 2 * DIM)
    ctx128 = context_table.reshape(VOCAB // 2, 2 * DIM)
    flat = _w2v(center_words.astype(jnp.int32), ctx_flat, cen128, ctx128)
    return flat.reshape(BATCH, NWORDS)


# speedup vs baseline: 1.2109x; 1.2109x over previous
import functools

import jax
import jax.numpy as jnp
from jax import lax
from jax.experimental import pallas as pl
from jax.experimental.pallas import tpu as pltpu
from jax.experimental.pallas import tpu_sc as plsc

VOCAB = 1_000_000
DIM = 64
BATCH = 16384
NWORDS = 20

NC = 2
NS = 16
NWK = NC * NS
BPW = BATCH // NWK
CB = 32
NCHUNK = BPW // CB
NPAIR = CB * NWORDS
IDX_SPLIT = 128
NSPLIT = NPAIR // IDX_SPLIT
L = 16
NJ = DIM // L


def _w2v_body(cen_words, ctx_words, cen_tab, ctx_tab, out,
              cidx_v, xidx_v, crow_v, xrow_v, cen_v, ctx_v, sco_v, sem):
    wid = lax.axis_index("s") * NC + lax.axis_index("c")
    base = wid * BPW
    iot = lax.iota(jnp.int32, L)

    def chunk(k, carry):
        cb = base + k * CB
        pltpu.sync_copy(cen_words.at[pl.ds(cb, CB)], cidx_v)
        pltpu.sync_copy(ctx_words.at[pl.ds(cb * NWORDS, NPAIR)], xidx_v)
        for t in range(CB // L):
            crow_v[pl.ds(t * L, L)] = cidx_v[pl.ds(t * L, L)] >> 1
        for t in range(NPAIR // L):
            xrow_v[pl.ds(t * L, L)] = xidx_v[pl.ds(t * L, L)] >> 1
        copies = [pltpu.async_copy(cen_tab.at[crow_v], cen_v, sem)]
        for j in range(NSPLIT):
            copies.append(pltpu.async_copy(
                ctx_tab.at[xrow_v.at[pl.ds(j * IDX_SPLIT, IDX_SPLIT)]],
                ctx_v.at[pl.ds(j * IDX_SPLIT, IDX_SPLIT)], sem))
        for c in copies:
            c.wait()
        def unit(u, carry2):
            b0 = u * 4
            accs = [jnp.zeros((L,), jnp.float32) for _ in range(5)]
            for i in range(4):
                b = b0 + i
                cpar = (plsc.load_gather(
                    cidx_v, [jnp.full((L,), b, jnp.int32)]) & 1) > 0
                cvs = [jnp.where(cpar,
                                 cen_v[b, pl.ds(DIM + j * L, L)],
                                 cen_v[b, pl.ds(j * L, L)])
                       for j in range(NJ)]
                for n in range(NWORDS):
                    row = b * NWORDS + n
                    xpar = (plsc.load_gather(
                        xidx_v, [jnp.full((L,), row, jnp.int32)]) & 1) > 0
                    p = jnp.where(xpar,
                                  ctx_v[row, pl.ds(DIM, L)],
                                  ctx_v[row, pl.ds(0, L)]) * cvs[0]
                    for j in range(1, NJ):
                        p = p + jnp.where(xpar,
                                          ctx_v[row, pl.ds(DIM + j * L, L)],
                                          ctx_v[row, pl.ds(j * L, L)]) * cvs[j]
                    s = jnp.sum(p)
                    fp = i * NWORDS + n
                    accs[fp // L] = jnp.where(
                        iot == (fp % L), jnp.full((L,), s, jnp.float32),
                        accs[fp // L])
            for g in range(5):
                sco_v[pl.ds(u * 80 + g * L, L)] = accs[g]
            return carry2

        lax.fori_loop(0, CB // 4, unit, 0)
        pltpu.sync_copy(sco_v, out.at[pl.ds(cb * NWORDS, NPAIR)])
        return carry

    lax.fori_loop(0, NCHUNK, chunk, 0)


_w2v = functools.partial(
    pl.kernel,
    mesh=plsc.VectorSubcoreMesh(core_axis_name="c", subcore_axis_name="s"),
    compiler_params=pltpu.CompilerParams(needs_layout_passes=False),
    out_type=jax.ShapeDtypeStruct((BATCH * NWORDS,), jnp.float32),
    scratch_types=[
        pltpu.VMEM((CB,), jnp.int32),
        pltpu.VMEM((NPAIR,), jnp.int32),
        pltpu.VMEM((CB,), jnp.int32),
        pltpu.VMEM((NPAIR,), jnp.int32),
        pltpu.VMEM((CB, 2 * DIM), jnp.float32),
        pltpu.VMEM((NPAIR, 2 * DIM), jnp.float32),
        pltpu.VMEM((NPAIR,), jnp.float32),
        pltpu.SemaphoreType.DMA,
    ],
)(_w2v_body)


@jax.jit
def kernel(center_words, context_words, center_table, context_table):
    ctx_flat = context_words.astype(jnp.int32).reshape(BATCH * NWORDS)
    cen128 = center_table.reshape(VOCAB // 2, 2 * DIM)
    ctx128 = context_table.reshape(VOCAB // 2, 2 * DIM)
    flat = _w2v(center_words.astype(jnp.int32), ctx_flat, cen128, ctx128)
    return flat.reshape(BATCH, NWORDS)
